# baseline (speedup 1.0000x reference)
import functools

import jax
import jax.numpy as jnp
from jax import lax
from jax.experimental import pallas as pl
from jax.experimental.pallas import tpu as pltpu
from jax.experimental.pallas import tpu_sc as plsc

L = 16
NPIX = 512 * 512
NCH = 64 * 3
NWORK = 32
CPW = NCH // NWORK
CHUNK = 8192
NCHUNK = NPIX // CHUNK
NPACK = NPIX // 4


def _tec_body(x_hbm, out_hbm, inbuf, outbuf, vbuf,
              hist0, hist1, hist2, hist3, lut,
              insem0, insem1, outsem0, outsem1):
  wid = lax.axis_index("s") * 2 + lax.axis_index("c")
  lanes = lax.iota(jnp.int32, L)
  ones = jnp.ones((L,), jnp.int32)
  mask = jnp.full((L,), 255, jnp.int32)
  hists = [hist0, hist1, hist2, hist3]

  def do_channel(i, _):
    c = wid * CPW + i
    cbase = c * NPIX

    def _zero(k, acc):
      z = jnp.zeros((L,), jnp.int32)
      for h in hists:
        h[pl.ds(k * L, L)] = z
      return acc + k
    zr = plsc.parallel_loop(0, 4096 // L, unroll=4,
                            carry=jnp.int32(0))(_zero)

    @pl.when(zr < 0)
    def _():
      hist0[pl.ds(0, L)] = jnp.zeros((L,), jnp.int32)

    def in_wait(slot, sem):
      pltpu.make_async_copy(x_hbm.at[pl.ds(0, CHUNK)], inbuf.at[slot],
                            sem).wait()

    def hist_slot(ck, slot):
      def _px(p, acc):
        base = p * (16 * L)
        wbase = ck * (CHUNK // 4) + p * (4 * L)
        for g in range(4):
          w = jnp.zeros((L,), jnp.int32)
          for u in range(4):
            xv = inbuf[slot, pl.ds(base + (4 * g + u) * L, L)]
            v = (xv * 255.0).astype(jnp.int32)
            plsc.addupdate_scatter(hists[u], [v * L + lanes], ones)
            w = w | (v << (8 * u))
          vbuf[pl.ds(wbase + g * L, L)] = w
        return acc + p
      hr = plsc.parallel_loop(0, CHUNK // (16 * L), unroll=4,
                              carry=jnp.int32(0))(_px)

      @pl.when(hr < 0)
      def _():
        vbuf[pl.ds(0, L)] = jnp.zeros((L,), jnp.int32)

    pltpu.async_copy(x_hbm.at[pl.ds(cbase, CHUNK)], inbuf.at[0], insem0)

    def hist_pair(k, _):
      ck = k * 2
      pltpu.async_copy(x_hbm.at[pl.ds(cbase + (ck + 1) * CHUNK, CHUNK)],
                       inbuf.at[1], insem1)
      in_wait(0, insem0)
      hist_slot(ck, 0)

      @pl.when(ck + 2 < NCHUNK)
      def _():
        pltpu.async_copy(x_hbm.at[pl.ds(cbase + (ck + 2) * CHUNK, CHUNK)],
                         inbuf.at[0], insem0)
      in_wait(1, insem1)
      hist_slot(ck + 1, 1)
      return 0
    lax.fori_loop(0, NCHUNK // 2, hist_pair, 0)

    carry = jnp.int32(0)
    key = jnp.full((L,), -1, jnp.int32)
    hvecs = []
    cums = []
    for j in range(16):
      base_idx = 256 * j + lanes * L
      h = jnp.zeros((L,), jnp.int32)
      for hr in hists:
        for l in range(16):
          h = h + plsc.load_gather(hr, [base_idx + l])
      cum = plsc.cumsum(h) + carry
      carry = carry + jnp.sum(h)
      bins = j * L + lanes
      key = jnp.maximum(key, jnp.where(h != 0, bins * 524288 + h, -1))
      hvecs.append(h)
      cums.append(cum)

    last_val = jnp.bitwise_and(jnp.max(key), 524287)
    step = lax.div(jnp.int32(NPIX) - last_val, jnp.int32(255))
    half = lax.div(step, jnp.int32(2))
    safe = jnp.maximum(step, jnp.int32(1))
    m = lax.div(jnp.int32(1 << 24), safe)
    identity = step == 0

    rcp_v = jnp.full((L,), m, jnp.int32).astype(jnp.float32) * jnp.float32(
        1.0 / (1 << 24))
    safe_v = jnp.full((L,), safe, jnp.int32)
    half_v = jnp.full((L,), half, jnp.int32)

    for j in range(16):
      a = cums[j] - hvecs[j] + half_v
      q = (a.astype(jnp.float32) * rcp_v).astype(jnp.int32)
      r = a - q * safe_v
      q = q - (r < 0).astype(jnp.int32)
      r = a - q * safe_v
      q = q + (r >= safe_v).astype(jnp.int32)
      q = jnp.clip(q, 0, 255)
      bins = j * L + lanes
      if j == 0:
        q = jnp.where(lanes == 0, 0, q)
      q = jnp.where(identity, bins, q)
      lut[pl.ds(j * L, L)] = q.astype(jnp.float32) * jnp.float32(1.0 / 255.0)

    def out_wait(slot, sem):
      pltpu.make_async_copy(outbuf.at[slot], out_hbm.at[pl.ds(0, CHUNK)],
                            sem).wait()

    def gather_slot(ck, slot):
      def _px(p, acc):
        base = p * (16 * L)
        wbase = ck * (CHUNK // 4) + p * (4 * L)
        for g in range(4):
          w = vbuf[pl.ds(wbase + g * L, L)]
          for u in range(4):
            v = (w >> (8 * u)) & mask
            outbuf[slot, pl.ds(base + (4 * g + u) * L, L)] = (
                plsc.load_gather(lut, [v]))
        return acc + p
      total = plsc.parallel_loop(0, CHUNK // (16 * L), unroll=4,
                                 carry=jnp.int32(0))(_px)

      @pl.when(total < 0)
      def _():
        outbuf[slot, pl.ds(0, L)] = jnp.zeros((L,), jnp.float32)
      pltpu.async_copy(outbuf.at[slot],
                       out_hbm.at[pl.ds(cbase + ck * CHUNK, CHUNK)],
                       outsem0 if slot == 0 else outsem1)

    def out_pair(k, _):
      ck = k * 2

      @pl.when(k >= 1)
      def _():
        out_wait(0, outsem0)
      gather_slot(ck, 0)

      @pl.when(k >= 1)
      def _():
        out_wait(1, outsem1)
      gather_slot(ck + 1, 1)
      return 0
    lax.fori_loop(0, NCHUNK // 2, out_pair, 0)

    out_wait(0, outsem0)
    out_wait(1, outsem1)
    return 0

  lax.fori_loop(0, CPW, do_channel, 0)


@jax.jit
def kernel(x):
  xf = x.reshape(NCH * NPIX)
  mesh = plsc.VectorSubcoreMesh(core_axis_name="c", subcore_axis_name="s")
  run = pl.kernel(
      _tec_body,
      out_type=jax.ShapeDtypeStruct((NCH * NPIX,), jnp.float32),
      mesh=mesh,
      scratch_types=[
          pltpu.VMEM((2, CHUNK), jnp.float32),
          pltpu.VMEM((2, CHUNK), jnp.float32),
          pltpu.VMEM((NPACK,), jnp.int32),
          pltpu.VMEM((4096,), jnp.int32),
          pltpu.VMEM((4096,), jnp.int32),
          pltpu.VMEM((4096,), jnp.int32),
          pltpu.VMEM((4096,), jnp.int32),
          pltpu.VMEM((256,), jnp.float32),
          pltpu.SemaphoreType.DMA,
          pltpu.SemaphoreType.DMA,
          pltpu.SemaphoreType.DMA,
          pltpu.SemaphoreType.DMA,
      ],
      compiler_params=pltpu.CompilerParams(needs_layout_passes=False,
                                           disable_bounds_checks=True),
  )
  return run(xf).reshape(x.shape)


# speedup vs baseline: 1.0256x; 1.0256x over previous
import functools

import jax
import jax.numpy as jnp
from jax import lax
from jax.experimental import pallas as pl
from jax.experimental.pallas import tpu as pltpu
from jax.experimental.pallas import tpu_sc as plsc

L = 16
NPIX = 512 * 512
NCH = 64 * 3
NWORK = 32
CPW = NCH // NWORK
CHUNK = 8192
NCHUNK = NPIX // CHUNK
NPACK = NPIX // 4


def _tec_body(x_hbm, out_hbm, inbuf, outbuf, vbuf,
              hist0, hist1, hist2, hist3, lut,
              insem0, insem1, outsem0, outsem1):
  wid = lax.axis_index("s") * 2 + lax.axis_index("c")
  lanes = lax.iota(jnp.int32, L)
  ones = jnp.ones((L,), jnp.int32)
  mask = jnp.full((L,), 255, jnp.int32)
  hists = [hist0, hist1, hist2, hist3]

  def do_channel(i, _):
    c = wid * CPW + i
    cbase = c * NPIX

    def _zero(k, acc):
      z = jnp.zeros((L,), jnp.int32)
      for h in hists:
        h[pl.ds(k * L, L)] = z
      return acc + k
    zr = plsc.parallel_loop(0, 4096 // L, unroll=2,
                            carry=jnp.int32(0))(_zero)

    @pl.when(zr < 0)
    def _():
      hist0[pl.ds(0, L)] = jnp.zeros((L,), jnp.int32)

    def in_wait(slot, sem):
      pltpu.make_async_copy(x_hbm.at[pl.ds(0, CHUNK)], inbuf.at[slot],
                            sem).wait()

    def hist_slot(ck, slot):
      def _px(p, acc):
        base = p * (16 * L)
        wbase = ck * (CHUNK // 4) + p * (4 * L)
        for g in range(4):
          w = jnp.zeros((L,), jnp.int32)
          for u in range(4):
            xv = inbuf[slot, pl.ds(base + (4 * g + u) * L, L)]
            v = (xv * 255.0).astype(jnp.int32)
            plsc.addupdate_scatter(hists[u], [v * L + lanes], ones)
            w = w | (v << (8 * u))
          vbuf[pl.ds(wbase + g * L, L)] = w
        return acc + p
      hr = plsc.parallel_loop(0, CHUNK // (16 * L), unroll=2,
                              carry=jnp.int32(0))(_px)

      @pl.when(hr < 0)
      def _():
        vbuf[pl.ds(0, L)] = jnp.zeros((L,), jnp.int32)

    pltpu.async_copy(x_hbm.at[pl.ds(cbase, CHUNK)], inbuf.at[0], insem0)

    def hist_pair(k, _):
      ck = k * 2
      pltpu.async_copy(x_hbm.at[pl.ds(cbase + (ck + 1) * CHUNK, CHUNK)],
                       inbuf.at[1], insem1)
      in_wait(0, insem0)
      hist_slot(ck, 0)

      @pl.when(ck + 2 < NCHUNK)
      def _():
        pltpu.async_copy(x_hbm.at[pl.ds(cbase + (ck + 2) * CHUNK, CHUNK)],
                         inbuf.at[0], insem0)
      in_wait(1, insem1)
      hist_slot(ck + 1, 1)
      return 0
    lax.fori_loop(0, NCHUNK // 2, hist_pair, 0)

    carry = jnp.int32(0)
    key = jnp.full((L,), -1, jnp.int32)
    hvecs = []
    cums = []
    for j in range(16):
      base_idx = 256 * j + lanes * L
      h = jnp.zeros((L,), jnp.int32)
      for hr in hists:
        for l in range(16):
          h = h + plsc.load_gather(hr, [base_idx + l])
      cum = plsc.cumsum(h) + carry
      carry = carry + jnp.sum(h)
      bins = j * L + lanes
      key = jnp.maximum(key, jnp.where(h != 0, bins * 524288 + h, -1))
      hvecs.append(h)
      cums.append(cum)

    last_val = jnp.bitwise_and(jnp.max(key), 524287)
    step = lax.div(jnp.int32(NPIX) - last_val, jnp.int32(255))
    half = lax.div(step, jnp.int32(2))
    safe = jnp.maximum(step, jnp.int32(1))
    m = lax.div(jnp.int32(1 << 24), safe)
    identity = step == 0

    rcp_v = jnp.full((L,), m, jnp.int32).astype(jnp.float32) * jnp.float32(
        1.0 / (1 << 24))
    safe_v = jnp.full((L,), safe, jnp.int32)
    half_v = jnp.full((L,), half, jnp.int32)

    for j in range(16):
      a = cums[j] - hvecs[j] + half_v
      q = (a.astype(jnp.float32) * rcp_v).astype(jnp.int32)
      r = a - q * safe_v
      q = q - (r < 0).astype(jnp.int32)
      r = a - q * safe_v
      q = q + (r >= safe_v).astype(jnp.int32)
      q = jnp.clip(q, 0, 255)
      bins = j * L + lanes
      if j == 0:
        q = jnp.where(lanes == 0, 0, q)
      q = jnp.where(identity, bins, q)
      lut[pl.ds(j * L, L)] = q.astype(jnp.float32) * jnp.float32(1.0 / 255.0)

    def out_wait(slot, sem):
      pltpu.make_async_copy(outbuf.at[slot], out_hbm.at[pl.ds(0, CHUNK)],
                            sem).wait()

    def gather_slot(ck, slot):
      def _px(p, acc):
        base = p * (16 * L)
        wbase = ck * (CHUNK // 4) + p * (4 * L)
        for g in range(4):
          w = vbuf[pl.ds(wbase + g * L, L)]
          for u in range(4):
            v = (w >> (8 * u)) & mask
            outbuf[slot, pl.ds(base + (4 * g + u) * L, L)] = (
                plsc.load_gather(lut, [v]))
        return acc + p
      total = plsc.parallel_loop(0, CHUNK // (16 * L), unroll=2,
                                 carry=jnp.int32(0))(_px)

      @pl.when(total < 0)
      def _():
        outbuf[slot, pl.ds(0, L)] = jnp.zeros((L,), jnp.float32)
      pltpu.async_copy(outbuf.at[slot],
                       out_hbm.at[pl.ds(cbase + ck * CHUNK, CHUNK)],
                       outsem0 if slot == 0 else outsem1)

    def out_pair(k, _):
      ck = k * 2

      @pl.when(k >= 1)
      def _():
        out_wait(0, outsem0)
      gather_slot(ck, 0)

      @pl.when(k >= 1)
      def _():
        out_wait(1, outsem1)
      gather_slot(ck + 1, 1)
      return 0
    lax.fori_loop(0, NCHUNK // 2, out_pair, 0)

    out_wait(0, outsem0)
    out_wait(1, outsem1)
    return 0

  lax.fori_loop(0, CPW, do_channel, 0)


@jax.jit
def kernel(x):
  xf = x.reshape(NCH * NPIX)
  mesh = plsc.VectorSubcoreMesh(core_axis_name="c", subcore_axis_name="s")
  run = pl.kernel(
      _tec_body,
      out_type=jax.ShapeDtypeStruct((NCH * NPIX,), jnp.float32),
      mesh=mesh,
      scratch_types=[
          pltpu.VMEM((2, CHUNK), jnp.float32),
          pltpu.VMEM((2, CHUNK), jnp.float32),
          pltpu.VMEM((NPACK,), jnp.int32),
          pltpu.VMEM((4096,), jnp.int32),
          pltpu.VMEM((4096,), jnp.int32),
          pltpu.VMEM((4096,), jnp.int32),
          pltpu.VMEM((4096,), jnp.int32),
          pltpu.VMEM((256,), jnp.float32),
          pltpu.SemaphoreType.DMA,
          pltpu.SemaphoreType.DMA,
          pltpu.SemaphoreType.DMA,
          pltpu.SemaphoreType.DMA,
      ],
      compiler_params=pltpu.CompilerParams(needs_layout_passes=False,
                                           disable_bounds_checks=True),
  )
  return run(xf).reshape(x.shape)
